# baseline (speedup 1.0000x reference)
import jax
import jax.numpy as jnp
from jax import lax
from jax.experimental import pallas as pl
from jax.experimental.pallas import tpu as pltpu
from jax.experimental.pallas import tpu_sc as plsc

MAX_LEN = 2048
D_MODEL = 4096
BATCH = 4
NC = 2
NS = 16
NW = NC * NS
POS_PER_W = MAX_LEN // NW
LANES = 16
NCHUNK = D_MODEL // LANES
NPHASE = 5
UNROLL = 8


def _body(in_hbm, tab_hbm, out_hbm, *scratch):
    rb = scratch[0:NPHASE]
    tb = scratch[NPHASE:2 * NPHASE]
    si = scratch[2 * NPHASE:3 * NPHASE]
    st = scratch[3 * NPHASE:4 * NPHASE]
    so = scratch[4 * NPHASE:5 * NPHASE]

    wid = lax.axis_index("s") * NC + lax.axis_index("c")
    l_base = wid * POS_PER_W

    def start_reads(g, p):
        l = l_base + jnp.minimum(g, POS_PER_W - 1)
        pltpu.async_copy(in_hbm.at[:, l], rb[p], si[p])
        pltpu.async_copy(tab_hbm.at[l], tb[p], st[p])

    def wait_read(p):
        pltpu.make_async_copy(in_hbm.at[:, 0], rb[p], si[p]).wait()
        pltpu.make_async_copy(tab_hbm.at[0], tb[p], st[p]).wait()

    def wait_write(p):
        pltpu.make_async_copy(rb[p], out_hbm.at[:, 0], so[p]).wait()

    for g in range(3):
        start_reads(g, g)

    def group_body(gg, carry):
        for p in range(NPHASE):
            g = gg * NPHASE + p
            pn = (p + 3) % NPHASE

            @pl.when(g + 3 < POS_PER_W + 1)
            def _():
                @pl.when(g >= 2)
                def _():
                    wait_write(pn)
                start_reads(jnp.minimum(g + 3, POS_PER_W - 1), pn)

            wait_read(p)

            @plsc.parallel_loop(0, NCHUNK, 1, unroll=UNROLL)
            def chunk_step(c, _p=p):
                o = c * LANES
                t = tb[_p][pl.ds(o, LANES)]
                for b in range(BATCH):
                    rb[_p][b, pl.ds(o, LANES)] = (
                        rb[_p][b, pl.ds(o, LANES)] + t
                    )

            pltpu.async_copy(
                rb[p],
                out_hbm.at[:, l_base + jnp.minimum(g, POS_PER_W - 1)],
                so[p])
        return carry

    lax.fori_loop(0, (POS_PER_W + NPHASE) // NPHASE, group_body, 0)

    for p in range(NPHASE):
        wait_write(p)


def kernel(inputs, pos_table):
    k = pl.kernel(
        _body,
        out_type=jax.ShapeDtypeStruct((BATCH, MAX_LEN, D_MODEL), jnp.float32),
        mesh=plsc.VectorSubcoreMesh(core_axis_name="c", subcore_axis_name="s"),
        scratch_types=(
            [pltpu.VMEM((BATCH, D_MODEL), jnp.float32) for _ in range(NPHASE)]
            + [pltpu.VMEM((D_MODEL,), jnp.float32) for _ in range(NPHASE)]
            + [pltpu.SemaphoreType.DMA for _ in range(3 * NPHASE)]
        ),
    )
    return k(inputs, pos_table)


# speedup vs baseline: 1.0120x; 1.0120x over previous
import jax
import jax.numpy as jnp
from jax import lax
from jax.experimental import pallas as pl
from jax.experimental.pallas import tpu as pltpu
from jax.experimental.pallas import tpu_sc as plsc

MAX_LEN = 2048
D_MODEL = 4096
BATCH = 4
NC = 2
NS = 16
NW = NC * NS
POS_PER_W = MAX_LEN // NW
LANES = 16
NCHUNK = D_MODEL // LANES
NPHASE = 4
UNROLL = 8


def _body(in_hbm, tab_hbm, out_hbm, *scratch):
    rb = scratch[0:NPHASE]
    tb = scratch[NPHASE:2 * NPHASE]
    si = scratch[2 * NPHASE:3 * NPHASE]
    st = scratch[3 * NPHASE:4 * NPHASE]
    so = scratch[4 * NPHASE:5 * NPHASE]

    wid = lax.axis_index("s") * NC + lax.axis_index("c")
    l_base = wid * POS_PER_W

    def start_reads(g, p):
        l = l_base + g
        pltpu.async_copy(in_hbm.at[:, l], rb[p], si[p])
        pltpu.async_copy(tab_hbm.at[l], tb[p], st[p])

    def wait_read(p):
        pltpu.make_async_copy(in_hbm.at[:, 0], rb[p], si[p]).wait()
        pltpu.make_async_copy(tab_hbm.at[0], tb[p], st[p]).wait()

    def wait_write(p):
        pltpu.make_async_copy(rb[p], out_hbm.at[:, 0], so[p]).wait()

    for g in range(2):
        start_reads(g, g)

    def group_body(gg, carry):
        for p in range(NPHASE):
            g = gg * NPHASE + p
            pn = (p + 2) % NPHASE

            @pl.when(g + 2 < POS_PER_W)
            def _():
                @pl.when(g >= 2)
                def _():
                    wait_write(pn)
                start_reads(g + 2, pn)

            wait_read(p)

            @plsc.parallel_loop(0, NCHUNK, 1, unroll=UNROLL)
            def chunk_step(c, _p=p):
                o = c * LANES
                t = tb[_p][pl.ds(o, LANES)]
                for b in range(BATCH):
                    rb[_p][b, pl.ds(o, LANES)] = (
                        rb[_p][b, pl.ds(o, LANES)] + t
                    )

            pltpu.async_copy(rb[p], out_hbm.at[:, l_base + g], so[p])
        return carry

    lax.fori_loop(0, POS_PER_W // NPHASE, group_body, 0)

    for p in range(NPHASE):
        wait_write(p)


def kernel(inputs, pos_table):
    k = pl.kernel(
        _body,
        out_type=jax.ShapeDtypeStruct((BATCH, MAX_LEN, D_MODEL), jnp.float32),
        mesh=plsc.VectorSubcoreMesh(core_axis_name="c", subcore_axis_name="s"),
        scratch_types=(
            [pltpu.VMEM((BATCH, D_MODEL), jnp.float32) for _ in range(NPHASE)]
            + [pltpu.VMEM((D_MODEL,), jnp.float32) for _ in range(NPHASE)]
            + [pltpu.SemaphoreType.DMA for _ in range(3 * NPHASE)]
        ),
    )
    return k(inputs, pos_table)
